# initial kernel scaffold (unmeasured)
import jax
import jax.numpy as jnp
from jax import lax
from jax.experimental import pallas as pl
from jax.experimental.pallas import tpu as pltpu


def kernel(
    x,
):
    def body(*refs):
        pass

    out_shape = jax.ShapeDtypeStruct(..., jnp.float32)
    return pl.pallas_call(body, out_shape=out_shape)(...)


# baseline (device time: 214349 ns/iter reference)
import jax
import jax.numpy as jnp
from jax import lax
from jax.experimental import pallas as pl
from jax.experimental.pallas import tpu as pltpu


def kernel(x):
    m, n = x.shape
    xb = x.astype(jnp.bfloat16)

    def body(x_ref, out_ref, recv_ref, send_sem, recv_sem):
        my_x = lax.axis_index("x")
        my_y = lax.axis_index("y")
        my_z = lax.axis_index("z")
        partner = (1 - my_x, my_y, my_z)

        barrier_sem = pltpu.get_barrier_semaphore()
        pl.semaphore_signal(
            barrier_sem,
            inc=1,
            device_id=partner,
            device_id_type=pl.DeviceIdType.MESH,
        )
        pl.semaphore_wait(barrier_sem, 1)

        rdma = pltpu.make_async_remote_copy(
            src_ref=x_ref,
            dst_ref=recv_ref,
            send_sem=send_sem,
            recv_sem=recv_sem,
            device_id=partner,
            device_id_type=pl.DeviceIdType.MESH,
        )
        rdma.start()
        rdma.wait()

        out_ref[...] = x_ref[...] + recv_ref[...]

    return pl.pallas_call(
        body,
        out_shape=jax.ShapeDtypeStruct((m, n), jnp.bfloat16),
        in_specs=[pl.BlockSpec(memory_space=pltpu.VMEM)],
        out_specs=pl.BlockSpec(memory_space=pltpu.VMEM),
        scratch_shapes=[
            pltpu.VMEM((m, n), jnp.bfloat16),
            pltpu.SemaphoreType.DMA,
            pltpu.SemaphoreType.DMA,
        ],
        compiler_params=pltpu.CompilerParams(collective_id=0),
    )(xb)


# device time: 138352 ns/iter; 1.5493x vs baseline; 1.5493x over previous
import jax
import jax.numpy as jnp
from jax import lax
from jax.experimental import pallas as pl
from jax.experimental.pallas import tpu as pltpu

K = 8


def kernel(x):
    m, n = x.shape
    half = m // 2
    ch = half // K
    xb = x.astype(jnp.bfloat16)

    def body(x_ref, out_ref, b_ref, sx, rx, sd, rd):
        my_x = lax.axis_index("x")
        my_y = lax.axis_index("y")
        my_z = lax.axis_index("z")
        partner = (1 - my_x, my_y, my_z)
        s = lax.rem(my_z, 2)
        domino = (my_x, my_y, my_z + 1 - 2 * s)

        my_off = s * half
        other_off = (1 - s) * half

        barrier_sem = pltpu.get_barrier_semaphore()
        for nbr in (partner, domino):
            pl.semaphore_signal(
                barrier_sem,
                inc=1,
                device_id=nbr,
                device_id_type=pl.DeviceIdType.MESH,
            )
        pl.semaphore_wait(barrier_sem, 2)

        x_rdmas = []
        for k in range(K):
            rows = pl.ds(my_off + k * ch, ch)
            r = pltpu.make_async_remote_copy(
                src_ref=x_ref.at[rows, :],
                dst_ref=b_ref.at[rows, :],
                send_sem=sx.at[k],
                recv_sem=rx.at[k],
                device_id=partner,
                device_id_type=pl.DeviceIdType.MESH,
            )
            r.start()
            x_rdmas.append(r)

        fwd_rdmas = []
        for k in range(K):
            x_rdmas[k].wait_recv()
            rows = pl.ds(my_off + k * ch, ch)
            f = pltpu.make_async_remote_copy(
                src_ref=b_ref.at[rows, :],
                dst_ref=b_ref.at[rows, :],
                send_sem=sd.at[k],
                recv_sem=rd.at[k],
                device_id=domino,
                device_id_type=pl.DeviceIdType.MESH,
            )
            f.start()
            fwd_rdmas.append(f)

        for k in range(K):
            rows = pl.ds(other_off + k * ch, ch)
            w = pltpu.make_async_remote_copy(
                src_ref=b_ref.at[rows, :],
                dst_ref=b_ref.at[rows, :],
                send_sem=sd.at[k],
                recv_sem=rd.at[k],
                device_id=domino,
                device_id_type=pl.DeviceIdType.MESH,
            )
            w.wait_recv()

        for r in x_rdmas:
            r.wait_send()
        for f in fwd_rdmas:
            f.wait_send()

        out_ref[...] = x_ref[...] + b_ref[...]

    return pl.pallas_call(
        body,
        out_shape=jax.ShapeDtypeStruct((m, n), jnp.bfloat16),
        in_specs=[pl.BlockSpec(memory_space=pltpu.VMEM)],
        out_specs=pl.BlockSpec(memory_space=pltpu.VMEM),
        scratch_shapes=[
            pltpu.VMEM((m, n), jnp.bfloat16),
            pltpu.SemaphoreType.DMA((K,)),
            pltpu.SemaphoreType.DMA((K,)),
            pltpu.SemaphoreType.DMA((K,)),
            pltpu.SemaphoreType.DMA((K,)),
        ],
        compiler_params=pltpu.CompilerParams(collective_id=0),
    )(xb)


# device time: 110682 ns/iter; 1.9366x vs baseline; 1.2500x over previous
import jax
import jax.numpy as jnp
from jax import lax
from jax.experimental import pallas as pl
from jax.experimental.pallas import tpu as pltpu

C = 4


def kernel(x):
    m, n = x.shape
    q = m // 4
    ch = q // C
    xp_rows = 688
    yp_rows = 672
    zp_rows = 688
    assert xp_rows + yp_rows + zp_rows == q

    xb = x.astype(jnp.bfloat16)

    def body(x_ref, out_ref, b_ref, sxs, rxs, sys_, rys, szs, rzs):
        my_x = lax.axis_index("x")
        my_y = lax.axis_index("y")
        my_z = lax.axis_index("z")
        sy = lax.rem(my_y, 2)
        sz = lax.rem(my_z, 2)
        partner = (1 - my_x, my_y, my_z)
        ydom = (my_x, my_y + 1 - 2 * sy, my_z)
        zdom = (my_x, my_y, my_z + 1 - 2 * sz)

        o_me = (2 * sy + sz) * q
        o_y = (2 * (1 - sy) + sz) * q
        o_z = (2 * sy + (1 - sz)) * q
        o_d = (2 * (1 - sy) + (1 - sz)) * q

        def mk(src_rows, dst_rows, ssem, rsem, dev):
            return pltpu.make_async_remote_copy(
                src_ref=b_ref.at[src_rows, :],
                dst_ref=b_ref.at[dst_rows, :],
                send_sem=ssem,
                recv_sem=rsem,
                device_id=dev,
                device_id_type=pl.DeviceIdType.MESH,
            )

        barrier_sem = pltpu.get_barrier_semaphore()
        for nbr in (partner, ydom, zdom):
            pl.semaphore_signal(
                barrier_sem,
                inc=1,
                device_id=nbr,
                device_id_type=pl.DeviceIdType.MESH,
            )
        pl.semaphore_wait(barrier_sem, 3)

        x_rd = []
        for k in range(C):
            rows = pl.ds(o_me + k * ch, ch)
            r = pltpu.make_async_remote_copy(
                src_ref=x_ref.at[rows, :],
                dst_ref=b_ref.at[rows, :],
                send_sem=sxs.at[k],
                recv_sem=rxs.at[k],
                device_id=partner,
                device_id_type=pl.DeviceIdType.MESH,
            )
            r.start()
            x_rd.append(r)
        rows_xp = pl.ds(o_d, xp_rows)
        r = pltpu.make_async_remote_copy(
            src_ref=x_ref.at[rows_xp, :],
            dst_ref=b_ref.at[rows_xp, :],
            send_sem=sxs.at[C],
            recv_sem=rxs.at[C],
            device_id=partner,
            device_id_type=pl.DeviceIdType.MESH,
        )
        r.start()
        x_rd.append(r)

        y1, z1 = [], []
        for k in range(C):
            x_rd[k].wait_recv()
            rows = pl.ds(o_me + k * ch, ch)
            f = mk(rows, rows, sys_.at[k], rys.at[k], ydom)
            f.start()
            y1.append(f)
            g = mk(rows, rows, szs.at[k], rzs.at[k], zdom)
            g.start()
            z1.append(g)

        yin = [
            mk(pl.ds(o_y + k * ch, ch), pl.ds(o_y + k * ch, ch),
               sys_.at[k], rys.at[k], ydom)
            for k in range(C)
        ]
        zin = [
            mk(pl.ds(o_z + k * ch, ch), pl.ds(o_z + k * ch, ch),
               szs.at[k], rzs.at[k], zdom)
            for k in range(C)
        ]

        for k in range(3):
            zin[k].wait_recv()
        rows_yp_src = pl.ds(o_z + xp_rows, yp_rows)
        ypo = mk(rows_yp_src, rows_yp_src, sys_.at[C], rys.at[C], ydom)
        ypo.start()

        for k in range(C):
            yin[k].wait_recv()
        rows_zp_src = pl.ds(o_y + xp_rows + yp_rows, zp_rows)
        zpo = mk(rows_zp_src, rows_zp_src, szs.at[C], rzs.at[C], zdom)
        zpo.start()

        x_rd[C].wait_recv()
        zin[3].wait_recv()
        rows_yp_in = pl.ds(o_d + xp_rows, yp_rows)
        mk(rows_yp_in, rows_yp_in, sys_.at[C], rys.at[C], ydom).wait_recv()
        rows_zp_in = pl.ds(o_d + xp_rows + yp_rows, zp_rows)
        mk(rows_zp_in, rows_zp_in, szs.at[C], rzs.at[C], zdom).wait_recv()

        for r in x_rd + y1 + z1 + [ypo, zpo]:
            r.wait_send()

        out_ref[...] = x_ref[...] + b_ref[...]

    return pl.pallas_call(
        body,
        out_shape=jax.ShapeDtypeStruct((m, n), jnp.bfloat16),
        in_specs=[pl.BlockSpec(memory_space=pltpu.VMEM)],
        out_specs=pl.BlockSpec(memory_space=pltpu.VMEM),
        scratch_shapes=[
            pltpu.VMEM((m, n), jnp.bfloat16),
            pltpu.SemaphoreType.DMA((C + 1,)),
            pltpu.SemaphoreType.DMA((C + 1,)),
            pltpu.SemaphoreType.DMA((C + 1,)),
            pltpu.SemaphoreType.DMA((C + 1,)),
            pltpu.SemaphoreType.DMA((C + 1,)),
            pltpu.SemaphoreType.DMA((C + 1,)),
        ],
        compiler_params=pltpu.CompilerParams(collective_id=0),
    )(xb)
